# baseline (speedup 1.0000x reference)
import functools

import jax
import jax.numpy as jnp
from jax import lax
from jax.experimental import pallas as pl
from jax.experimental.pallas import tpu as pltpu
from jax.experimental.pallas import tpu_sc as plsc

POS = 200
BATCH = 1024
EMB = 64
FLAT = POS * BATCH

NC = 2
NS = 16
NW = NC * NS

TOK_PER_W = FLAT // NW
GCHUNK = 100
BLK = 200
CH_PER_BLK = BLK // GCHUNK
BLK_PER_W = TOK_PER_W // BLK
CH_PER_W = BLK_PER_W * CH_PER_BLK
NBUF = 4
LEAD = 2


def _sc_body(tokens_hbm, table_hbm, out_hbm, idx_v, *rest):
    bufs = rest[:NBUF]
    gsems = rest[NBUF:2 * NBUF]
    osems = rest[2 * NBUF:3 * NBUF]

    cid = lax.axis_index("c")
    sid = lax.axis_index("s")
    wid = cid * NS + sid

    pltpu.sync_copy(tokens_hbm.at[wid], idx_v)

    def gather_descs(blk, b):
        ch = blk * CH_PER_BLK
        return [
            pltpu.make_async_copy(
                table_hbm.at[idx_v.at[ch]],
                bufs[b].at[pl.ds(0, GCHUNK)], gsems[b]),
            pltpu.make_async_copy(
                table_hbm.at[idx_v.at[ch + 1]],
                bufs[b].at[pl.ds(GCHUNK, GCHUNK)], gsems[b]),
        ]

    def out_desc(blk, b):
        base = wid * TOK_PER_W + blk * BLK
        return pltpu.make_async_copy(
            bufs[b], out_hbm.at[pl.ds(base, BLK)], osems[b])

    for t in range(LEAD):
        for d in gather_descs(t, t % NBUF):
            d.start()

    def turn(t, b):
        nxt = t + LEAD

        @pl.when(nxt < BLK_PER_W)
        def _():
            bb = (b + LEAD) % NBUF

            @pl.when(nxt >= NBUF)
            def _():
                out_desc(nxt - NBUF, bb).wait()
            for d in gather_descs(nxt, bb):
                d.start()

        for d in gather_descs(t, b):
            d.wait()
        out_desc(t, b).start()

    def outer(g, carry):
        for b in range(NBUF):
            turn(g * NBUF + b, b)
        return carry

    lax.fori_loop(0, BLK_PER_W // NBUF, outer, 0)

    for t in range(BLK_PER_W - NBUF, BLK_PER_W):
        out_desc(t, t % NBUF).wait()


def _pe_body(x_ref, pe_ref, o_ref):
    s = pl.program_id(0)
    o_ref[0] = jnp.transpose(x_ref[0], (1, 0)) + pe_ref[s][:, None]


@jax.jit
def _run(tokens_r, table, pe2):
    gather = pl.kernel(
        _sc_body,
        out_type=jax.ShapeDtypeStruct((FLAT, EMB), jnp.float32),
        mesh=plsc.VectorSubcoreMesh(core_axis_name="c", subcore_axis_name="s"),
        scratch_types=(
            [pltpu.VMEM((CH_PER_W, GCHUNK), jnp.int32)]
            + [pltpu.VMEM((BLK, EMB), jnp.float32)] * NBUF
            + [pltpu.SemaphoreType.DMA] * (2 * NBUF)
        ),
        compiler_params=pltpu.CompilerParams(use_tc_tiling_on_sc=False),
    )
    raw = gather(tokens_r, table).reshape(POS, BATCH, EMB)

    out = pl.pallas_call(
        _pe_body,
        grid=(POS,),
        in_specs=[
            pl.BlockSpec((1, BATCH, EMB), lambda s: (s, 0, 0)),
            pl.BlockSpec((POS, EMB), lambda s: (0, 0)),
        ],
        out_specs=pl.BlockSpec((1, EMB, BATCH), lambda s: (s, 0, 0)),
        out_shape=jax.ShapeDtypeStruct((POS, EMB, BATCH), jnp.float32),
    )(raw, pe2)
    return out


def kernel(tokens, embedding_table, positional_encoding):
    seq = tokens.shape[1]
    tokens_r = tokens.T.reshape(NW, CH_PER_W, GCHUNK)
    pe2 = positional_encoding[:seq]
    out = _run(tokens_r, embedding_table, pe2)
    return jnp.transpose(out, (2, 0, 1))


# speedup vs baseline: 1.0144x; 1.0144x over previous
import functools

import jax
import jax.numpy as jnp
from jax import lax
from jax.experimental import pallas as pl
from jax.experimental.pallas import tpu as pltpu
from jax.experimental.pallas import tpu_sc as plsc

POS = 200
BATCH = 1024
EMB = 64
FLAT = POS * BATCH

NC = 2
NS = 16
NW = NC * NS

SHI = POS // 8
BHI = BATCH // 128
UNITS = SHI * BHI
NBUF = 4

MAXU = 7


def _sc_body(tok_hbm, pe_hbm, table_hbm, out_hbm, idx_v, pe_v, *rest):
    bufs = rest[:NBUF]
    gsems = rest[NBUF:2 * NBUF]
    osems = rest[2 * NBUF:3 * NBUF]

    cid = lax.axis_index("c")
    sid = lax.axis_index("s")
    wid = cid * NS + sid

    ustart = jnp.where(wid < 8, 7 * wid, 56 + 6 * (wid - 8))
    ucnt = jnp.where(wid < 8, 7, 6)

    def load_tok(k, c):
        u = ustart + k
        pltpu.sync_copy(tok_hbm.at[u // BHI, u % BHI], idx_v.at[k])
        return c
    lax.fori_loop(0, ucnt, load_tok, 0)
    pltpu.sync_copy(pe_hbm, pe_v)

    def gather_descs(k, kblk, b):
        return [
            pltpu.make_async_copy(
                table_hbm.at[idx_v.at[k, 2 * kblk + h]],
                bufs[b].at[pl.ds(128 * h, 128)], gsems[b])
            for h in range(2)
        ]

    def out_descs(u, kblk, b):
        shi = u // BHI
        bhi = u % BHI
        descs = []
        for h in range(2):
            f0 = (8 * shi + 2 * kblk + h) * BATCH + 128 * bhi
            descs.append(pltpu.make_async_copy(
                bufs[b].at[pl.ds(128 * h, 128)],
                out_hbm.at[pl.ds(f0, 128)], osems[b]))
        return descs

    def unit(k):
        u = ustart + k
        for kblk in range(NBUF):
            @pl.when(k > 0)
            def _():
                for d in out_descs(u - 1, kblk, kblk):
                    d.wait()
            for d in gather_descs(k, kblk, kblk):
                d.start()
        for kblk in range(NBUF):
            for d in gather_descs(k, kblk, kblk):
                d.wait()
            for h in range(2):
                s = 8 * (u // BHI) + 2 * kblk + h

                def add_row(r, c2):
                    for cc in range(EMB // 16):
                        sl = pl.ds(cc * 16, 16)
                        plsc.addupdate(bufs[kblk].at[128 * h + r, sl],
                                       pe_v[s, sl])
                    return c2
                lax.fori_loop(0, 128, add_row, 0, unroll=4)
            for d in out_descs(u, kblk, kblk):
                d.start()

    def uloop(k, c):
        unit(k)
        return c
    lax.fori_loop(0, ucnt, uloop, 0)

    for kblk in range(NBUF):
        for d in out_descs(ustart + ucnt - 1, kblk, kblk):
            d.wait()


@jax.jit
def _run(tok4, pe2, table):
    gather = pl.kernel(
        _sc_body,
        out_type=jax.ShapeDtypeStruct((FLAT, EMB), jnp.float32),
        mesh=plsc.VectorSubcoreMesh(core_axis_name="c", subcore_axis_name="s"),
        scratch_types=(
            [pltpu.VMEM((MAXU, 8, 128), jnp.int32),
             pltpu.VMEM((POS, EMB), jnp.float32)]
            + [pltpu.VMEM((256, EMB), jnp.float32)] * NBUF
            + [pltpu.SemaphoreType.DMA] * (2 * NBUF)
        ),
        compiler_params=pltpu.CompilerParams(use_tc_tiling_on_sc=False),
    )
    return gather(tok4, pe2, table)


def kernel(tokens, embedding_table, positional_encoding):
    seq = tokens.shape[1]
    tok4 = (tokens.T
            .reshape(SHI, 8, BHI, 128)
            .transpose(0, 2, 1, 3))
    pe2 = positional_encoding[:seq]
    raw = _run(tok4, pe2, embedding_table)
    return jnp.transpose(raw.reshape(POS, BATCH, EMB), (1, 0, 2))


# speedup vs baseline: 1.0166x; 1.0021x over previous
import functools

import jax
import jax.numpy as jnp
from jax import lax
from jax.experimental import pallas as pl
from jax.experimental.pallas import tpu as pltpu
from jax.experimental.pallas import tpu_sc as plsc

POS = 200
BATCH = 1024
EMB = 64
FLAT = POS * BATCH

NC = 2
NS = 16
NW = NC * NS

SHI = POS // 8
BHI = BATCH // 128
UNITS = SHI * BHI
NBUF = 4

MAXU = 7


def _sc_body(tok_hbm, pe_hbm, table_hbm, out_hbm, idx_v, pe_v, *rest):
    bufs = rest[:NBUF]
    gsems = rest[NBUF:2 * NBUF]
    osems = rest[2 * NBUF:3 * NBUF]

    cid = lax.axis_index("c")
    sid = lax.axis_index("s")
    wid = cid * NS + sid

    ustart = jnp.where(wid < 8, 7 * wid, 56 + 6 * (wid - 8))
    ucnt = jnp.where(wid < 8, 7, 6)

    def load_tok(k, c):
        u = ustart + k
        pltpu.sync_copy(
            tok_hbm.at[pl.ds(8 * (u // BHI), 8), pl.ds(128 * (u % BHI), 128)],
            idx_v.at[k])
        return c
    lax.fori_loop(0, ucnt, load_tok, 0)
    pltpu.sync_copy(pe_hbm, pe_v)

    def gather_descs(k, kblk, b):
        return [
            pltpu.make_async_copy(
                table_hbm.at[idx_v.at[k, 2 * kblk + h]],
                bufs[b].at[pl.ds(128 * h, 128)], gsems[b])
            for h in range(2)
        ]

    def out_descs(u, kblk, b):
        shi = u // BHI
        bhi = u % BHI
        descs = []
        for h in range(2):
            s = 8 * shi + 2 * kblk + h
            descs.append(pltpu.make_async_copy(
                bufs[b].at[pl.ds(128 * h, 128)],
                out_hbm.at[s, pl.ds(128 * bhi, 128)], osems[b]))
        return descs

    def unit(k):
        u = ustart + k
        for kblk in range(NBUF):
            @pl.when(k > 0)
            def _():
                for d in out_descs(u - 1, kblk, kblk):
                    d.wait()
            for d in gather_descs(k, kblk, kblk):
                d.start()
        for kblk in range(NBUF):
            for d in gather_descs(k, kblk, kblk):
                d.wait()
            for h in range(2):
                s = 8 * (u // BHI) + 2 * kblk + h

                def add_row(r, c2):
                    for cc in range(EMB // 16):
                        sl = pl.ds(cc * 16, 16)
                        plsc.addupdate(bufs[kblk].at[128 * h + r, sl],
                                       pe_v[s, sl])
                    return c2
                lax.fori_loop(0, 128, add_row, 0, unroll=4)
            for d in out_descs(u, kblk, kblk):
                d.start()

    def uloop(k, c):
        unit(k)
        return c
    lax.fori_loop(0, ucnt, uloop, 0)

    for kblk in range(NBUF):
        for d in out_descs(ustart + ucnt - 1, kblk, kblk):
            d.wait()


@jax.jit
def _run(tok4, pe2, table):
    gather = pl.kernel(
        _sc_body,
        out_type=jax.ShapeDtypeStruct((POS, BATCH, EMB), jnp.float32),
        mesh=plsc.VectorSubcoreMesh(core_axis_name="c", subcore_axis_name="s"),
        scratch_types=(
            [pltpu.VMEM((MAXU, 8, 128), jnp.int32),
             pltpu.VMEM((POS, EMB), jnp.float32)]
            + [pltpu.VMEM((256, EMB), jnp.float32)] * NBUF
            + [pltpu.SemaphoreType.DMA] * (2 * NBUF)
        ),
        compiler_params=pltpu.CompilerParams(use_tc_tiling_on_sc=False),
    )
    return gather(tok4, pe2, table)


def kernel(tokens, embedding_table, positional_encoding):
    seq = tokens.shape[1]
    tok_t = tokens.T
    pe2 = positional_encoding[:seq]
    raw = _run(tok_t, pe2, embedding_table)
    return jnp.transpose(raw, (1, 0, 2))
